# baseline (speedup 1.0000x reference)
import functools

import jax
import jax.numpy as jnp
from jax import lax
from jax.experimental import pallas as pl
from jax.experimental.pallas import tpu as pltpu
from jax.experimental.pallas import tpu_sc as plsc

B = 16384
P = 4
T = 81
C = 39
D = T * C
L = 16
NC = 2
NS = 16
NW = NC * NS
RW = B // NW
RB = 16
NBLK = RW // RB
RSUB = 4
KFULL = D // L
TAILOFF = D - L
TAILSKIP = L - (D - KFULL * L)


def _sc_body(x_hbm, p_hbm, y_hbm, a_hbm,
             pbuf, xbuf0, xbuf1, ybuf, abuf, accbuf,
             sem0, sem1):
    wid = lax.axis_index("s") * NC + lax.axis_index("c")
    base = wid * RW

    pltpu.async_copy(x_hbm.at[pl.ds(base, RB)], xbuf0, sem0)
    pltpu.async_copy(x_hbm.at[pl.ds(base + RB, RB)], xbuf1, sem1)
    pltpu.sync_copy(p_hbm, pbuf)

    zero = jnp.zeros((L,), jnp.float32)
    lane = lax.iota(jnp.int32, L)
    tmask = lane >= TAILSKIP

    def pnorm_body(k, accs):
        out = []
        for p in range(P):
            pv = pbuf[p, pl.ds(k * L, L)]
            out.append(accs[p] + pv * pv)
        return tuple(out)

    paccs = lax.fori_loop(0, KFULL, pnorm_body, (zero,) * P)
    for p in range(P):
        pv = pbuf[p, pl.ds(TAILOFF, L)]
        accbuf[p, :] = paccs[p] + jnp.where(tmask, pv * pv, 0.0)
    ptot = zero
    for c in range(L):
        ptot = ptot + plsc.load_gather(
            accbuf, [lane, jnp.full((L,), c, jnp.int32)])
    accbuf[P, :] = ptot
    pps = [plsc.load_gather(accbuf, [jnp.full((L,), P, jnp.int32),
                                     jnp.full((L,), p, jnp.int32)])
           for p in range(P)]

    xbufs = (xbuf0, xbuf1)
    sems = (sem0, sem1)

    def block_body(i, carry):
        for b in range(2):
            g = i * 2 + b
            xb = xbufs[b]
            sem = sems[b]
            pltpu.make_async_copy(x_hbm.at[pl.ds(base, RB)], xb, sem).wait()

            for s in range(RB // RSUB):
                def kbody(k, accs):
                    pvs = [pbuf[p, pl.ds(k * L, L)] for p in range(P)]
                    out = []
                    for j in range(RSUB):
                        xv = xb[s * RSUB + j, pl.ds(k * L, L)]
                        a = j * (P + 1)
                        out.append(accs[a] + xv * xv)
                        for p in range(P):
                            out.append(accs[a + 1 + p] + xv * pvs[p])
                    return tuple(out)

                accs = list(lax.fori_loop(0, KFULL, kbody,
                                          (zero,) * (RSUB * (P + 1))))
                pvs = [pbuf[p, pl.ds(TAILOFF, L)] for p in range(P)]
                for j in range(RSUB):
                    xv = xb[s * RSUB + j, pl.ds(TAILOFF, L)]
                    a = j * (P + 1)
                    accs[a] = accs[a] + jnp.where(tmask, xv * xv, 0.0)
                    for p in range(P):
                        accs[a + 1 + p] = (accs[a + 1 + p]
                                           + jnp.where(tmask, xv * pvs[p], 0.0))
                for j in range(RSUB):
                    r = s * RSUB + j
                    a = j * (P + 1)
                    for st in range(P + 1):
                        accbuf[st * L + r, :] = accs[a + st]

            @pl.when(g + 2 < NBLK)
            def _():
                pltpu.async_copy(
                    x_hbm.at[pl.ds(base + (g + 2) * RB, RB)], xb, sem)

            sums = []
            for st in range(P + 1):
                rowidx = lane + st * L
                tot = zero
                for c in range(L):
                    colidx = jnp.full((L,), c, jnp.int32)
                    tot = tot + plsc.load_gather(accbuf, [rowidx, colidx])
                sums.append(tot)
            xx = sums[0]
            ys = [xx - 2.0 * sums[1 + p] + pps[p] for p in range(P)]

            m = ys[0]
            am = jnp.zeros((L,), jnp.int32)
            for p in range(1, P):
                lt = ys[p] < m
                am = jnp.where(lt, p, am)
                m = jnp.where(lt, ys[p], m)

            rowg = g * RB + lane
            for p in range(P):
                plsc.store_scatter(ybuf, [rowg, jnp.full((L,), p, jnp.int32)],
                                   ys[p])
            plsc.store_scatter(abuf, [rowg], am)
        return carry

    lax.fori_loop(0, NBLK // 2, block_body, 0)

    pltpu.sync_copy(ybuf, y_hbm.at[pl.ds(base, RW)])
    pltpu.sync_copy(abuf, a_hbm.at[pl.ds(base, RW)])


@functools.lru_cache(maxsize=1)
def _build_sc_call():
    mesh = plsc.VectorSubcoreMesh(core_axis_name="c", subcore_axis_name="s",
                                  num_cores=NC, num_subcores=NS)
    return functools.partial(
        pl.kernel,
        out_type=(jax.ShapeDtypeStruct((B, P), jnp.float32),
                  jax.ShapeDtypeStruct((B,), jnp.int32)),
        mesh=mesh,
        scratch_types=[
            pltpu.VMEM((P, D), jnp.float32),
            pltpu.VMEM((RB, D), jnp.float32),
            pltpu.VMEM((RB, D), jnp.float32),
            pltpu.VMEM((RW, P), jnp.float32),
            pltpu.VMEM((RW,), jnp.int32),
            pltpu.VMEM(((P + 1) * L, L), jnp.float32),
            pltpu.SemaphoreType.DMA,
            pltpu.SemaphoreType.DMA,
        ],
        compiler_params=pltpu.CompilerParams(needs_layout_passes=False,
                                             use_tc_tiling_on_sc=False),
    )(_sc_body)


def kernel(x, prototypes):
    x2 = x.reshape(B, D)
    p2 = prototypes.reshape(P, D)
    y, am = _build_sc_call()(x2, p2)
    return (y, am)
